# baseline (speedup 1.0000x reference)
import functools

import jax
import jax.numpy as jnp
from jax import lax
from jax.experimental import pallas as pl
from jax.experimental.pallas import tpu as pltpu
from jax.experimental.pallas import tpu_sc as plsc

ROWS = 128
N = 32768
NC = 2
NS = 16
NW = NC * NS
RPW = ROWS // NW
NV = N // 16
RADIX = 256
NPASS = 4
C = 8
CLEN = N // C
CSH = 12
CV = NV // C
SIGN = -2147483648


def _digit(k, shift):
  return lax.shift_right_logical(k, shift) & (RADIX - 1)


def _sort_body(x_hbm, out_hbm, bufa, bufb, hist, offs):
  wid = lax.axis_index("s") * NC + lax.axis_index("c")

  def scan_hist():
    z = jnp.zeros((16,), jnp.int32)
    carry = jnp.int32(0)
    for v in range(RADIX // 16):
      hs = [hist[pl.ds(c * RADIX + v * 16, 16)] for c in range(C)]
      tot = hs[0]
      for c in range(1, C):
        tot = tot + hs[c]
      run = plsc.cumsum(tot) - tot + carry
      carry = carry + jnp.sum(tot)
      for c in range(C):
        hist[pl.ds(c * RADIX + v * 16, 16)] = z
        offs[pl.ds(c * RADIX + v * 16, 16)] = run
        run = run + hs[c]

  def do_row(row, _):
    pltpu.sync_copy(x_hbm.at[row], bufa)

    @plsc.parallel_loop(0, C * RADIX // 16, unroll=8)
    def _(i):
      hist[pl.ds(i * 16, 16)] = jnp.zeros((16,), jnp.int32)

    def h0(i, _):
      @plsc.parallel_loop(0, C, unroll=C)
      def _(c):
        k = bufa[pl.ds(c * CLEN + i * 16, 16)]
        k = k ^ (lax.shift_right_arithmetic(k, 31) | SIGN)
        bufa[pl.ds(c * CLEN + i * 16, 16)] = k
        d = _digit(k, 0) + c * RADIX
        cnt, last = plsc.scan_count(d)
        plsc.addupdate_scatter(hist, [d], cnt, mask=last)

      return 0

    lax.fori_loop(0, CV, h0, 0)

    for p in range(NPASS):
      src, dst = (bufa, bufb) if p % 2 == 0 else (bufb, bufa)
      shift = 8 * p
      scan_hist()
      final = p == NPASS - 1

      def perm(i, _, src=src, dst=dst, shift=shift, final=final):
        @plsc.parallel_loop(0, C, unroll=(C if final else C // 2))
        def _(c):
          k = src[pl.ds(c * CLEN + i * 16, 16)]
          d = _digit(k, shift) + c * RADIX
          cnt, last = plsc.scan_count(d)
          pos = plsc.load_gather(offs, [d]) + cnt
          plsc.store_scatter(offs, [d], pos, mask=last)
          pos = pos - 1
          if final:
            out = k ^ (~lax.shift_right_arithmetic(k, 31) | SIGN)
          else:
            d2 = _digit(k, shift + 8) + (
                lax.shift_right_logical(pos, CSH) * RADIX)
            cnt2, last2 = plsc.scan_count(d2)
            plsc.addupdate_scatter(hist, [d2], cnt2, mask=last2)
            out = k
          plsc.store_scatter(dst, [pos], out)

        return 0

      lax.fori_loop(0, CV, perm, 0)

    final_buf = bufb if NPASS % 2 == 1 else bufa
    pltpu.sync_copy(final_buf, out_hbm.at[row])
    return 0

  lax.fori_loop(wid * RPW, (wid + 1) * RPW, do_row, 0)


@jax.jit
def kernel(x):
  mesh = plsc.VectorSubcoreMesh(
      core_axis_name="c", subcore_axis_name="s", num_cores=NC, num_subcores=NS
  )
  run = pl.kernel(
      _sort_body,
      out_type=jax.ShapeDtypeStruct((ROWS, N), jnp.int32),
      mesh=mesh,
      scratch_types=[
          pltpu.VMEM((N,), jnp.int32),
          pltpu.VMEM((N,), jnp.int32),
          pltpu.VMEM((C * RADIX,), jnp.int32),
          pltpu.VMEM((C * RADIX,), jnp.int32),
      ],
      compiler_params=pltpu.CompilerParams(needs_layout_passes=False),
  )
  out_i32 = run(lax.bitcast_convert_type(x, jnp.int32))
  return lax.bitcast_convert_type(out_i32, jnp.float32)


# speedup vs baseline: 1.6612x; 1.6612x over previous
import functools

import jax
import jax.numpy as jnp
from jax import lax
from jax.experimental import pallas as pl
from jax.experimental.pallas import tpu as pltpu
from jax.experimental.pallas import tpu_sc as plsc

ROWS = 128
N = 32768
NC = 2
NS = 16
NW = NC * NS
RPW = ROWS // NW
NV = N // 16
RADIX = 256
NPASS = 4
C = 16
CLEN = N // C
CV = NV // C
SIGN = -2147483648


def _digit(k, shift):
  return lax.shift_right_logical(k, shift) & (RADIX - 1)


def _sort_body(x_hbm, out_hbm, bufa, bufb, hist, offs):
  wid = lax.axis_index("s") * NC + lax.axis_index("c")

  def zero_hist():
    z = jnp.zeros((16,), jnp.int32)

    def zh(i, _):
      hist[pl.ds(i * 16, 16)] = z
      return 0

    lax.fori_loop(0, C * RADIX // 16, zh, 0)

  def scan_hist():
    carry = jnp.int32(0)
    for v in range(RADIX // 16):
      hs = [hist[pl.ds(c * RADIX + v * 16, 16)] for c in range(C)]
      tot = hs[0]
      for c in range(1, C):
        tot = tot + hs[c]
      run = plsc.cumsum(tot) - tot + carry
      carry = carry + jnp.sum(tot)
      for c in range(C):
        offs[pl.ds(c * RADIX + v * 16, 16)] = run
        run = run + hs[c]

  def do_row(row, _):
    pltpu.sync_copy(x_hbm.at[row], bufa)

    zero_hist()

    def h0(i, _):
      @plsc.parallel_loop(0, C, unroll=C // 2)
      def _(c):
        k = bufa[pl.ds(c * CLEN + i * 16, 16)]
        k = k ^ (lax.shift_right_arithmetic(k, 31) | SIGN)
        bufa[pl.ds(c * CLEN + i * 16, 16)] = k
        d = _digit(k, 0) + c * RADIX
        cnt, last = plsc.scan_count(d)
        plsc.addupdate_scatter(hist, [d], cnt, mask=last)

      return 0

    lax.fori_loop(0, CV, h0, 0)

    for p in range(NPASS):
      src, dst = (bufa, bufb) if p % 2 == 0 else (bufb, bufa)
      shift = 8 * p
      scan_hist()
      final = p == NPASS - 1

      def perm(i, _, src=src, dst=dst, shift=shift, final=final):
        @plsc.parallel_loop(0, C, unroll=C // 2)
        def _(c):
          k = src[pl.ds(c * CLEN + i * 16, 16)]
          d = _digit(k, shift) + c * RADIX
          cnt, last = plsc.scan_count(d)
          pos = plsc.load_gather(offs, [d]) + cnt
          plsc.store_scatter(offs, [d], pos, mask=last)
          pos = pos - 1
          if final:
            out = k ^ (~lax.shift_right_arithmetic(k, 31) | SIGN)
          else:
            out = k
          plsc.store_scatter(dst, [pos], out)

        return 0

      lax.fori_loop(0, CV, perm, 0)

      if not final:
        zero_hist()
        shift2 = 8 * (p + 1)

        def hist_next(i, _, dst=dst, shift2=shift2):
          @plsc.parallel_loop(0, C, unroll=C // 2)
          def _(c):
            k = dst[pl.ds(c * CLEN + i * 16, 16)]
            d = _digit(k, shift2) + c * RADIX
            cnt, last = plsc.scan_count(d)
            plsc.addupdate_scatter(hist, [d], cnt, mask=last)

          return 0

        lax.fori_loop(0, CV, hist_next, 0)

    final_buf = bufb if NPASS % 2 == 1 else bufa
    pltpu.sync_copy(final_buf, out_hbm.at[row])
    return 0

  lax.fori_loop(wid * RPW, (wid + 1) * RPW, do_row, 0)


@jax.jit
def kernel(x):
  mesh = plsc.VectorSubcoreMesh(
      core_axis_name="c", subcore_axis_name="s", num_cores=NC, num_subcores=NS
  )
  run = pl.kernel(
      _sort_body,
      out_type=jax.ShapeDtypeStruct((ROWS, N), jnp.int32),
      mesh=mesh,
      scratch_types=[
          pltpu.VMEM((N,), jnp.int32),
          pltpu.VMEM((N,), jnp.int32),
          pltpu.VMEM((C * RADIX,), jnp.int32),
          pltpu.VMEM((C * RADIX,), jnp.int32),
      ],
      compiler_params=pltpu.CompilerParams(needs_layout_passes=False),
  )
  out_i32 = run(lax.bitcast_convert_type(x, jnp.int32))
  return lax.bitcast_convert_type(out_i32, jnp.float32)


# speedup vs baseline: 1.9789x; 1.1912x over previous
import functools

import jax
import jax.numpy as jnp
from jax import lax
from jax.experimental import pallas as pl
from jax.experimental.pallas import tpu as pltpu
from jax.experimental.pallas import tpu_sc as plsc

ROWS = 128
N = 32768
NC = 2
NS = 16
NW = NC * NS
RPW = ROWS // NW
NV = N // 16
RADIX = 256
NPASS = 4
C = 8
CLEN = N // C
CV = NV // C
SIGN = -2147483648


def _digit(k, shift):
  return lax.shift_right_logical(k, shift) & (RADIX - 1)


def _sort_body(x_hbm, out_hbm, bufa, bufb, hist, offs):
  wid = lax.axis_index("s") * NC + lax.axis_index("c")

  def zero_hist():
    z = jnp.zeros((16,), jnp.int32)

    def zh(i, _):
      hist[pl.ds(i * 16, 16)] = z
      return 0

    lax.fori_loop(0, C * RADIX // 16, zh, 0)

  def scan_hist():
    carry = jnp.int32(0)
    for v in range(RADIX // 16):
      hs = [hist[pl.ds(c * RADIX + v * 16, 16)] for c in range(C)]
      tot = hs[0]
      for c in range(1, C):
        tot = tot + hs[c]
      run = plsc.cumsum(tot) - tot + carry
      carry = carry + jnp.sum(tot)
      for c in range(C):
        offs[pl.ds(c * RADIX + v * 16, 16)] = run
        run = run + hs[c]

  def do_row(row, _):
    pltpu.sync_copy(x_hbm.at[row], bufa)

    zero_hist()

    def h0(i, _):
      @plsc.parallel_loop(0, C, unroll=C)
      def _(c):
        k = bufa[pl.ds(c * CLEN + i * 16, 16)]
        k = k ^ (lax.shift_right_arithmetic(k, 31) | SIGN)
        bufa[pl.ds(c * CLEN + i * 16, 16)] = k
        d = _digit(k, 0) + c * RADIX
        cnt, last = plsc.scan_count(d)
        plsc.addupdate_scatter(hist, [d], cnt, mask=last)

      return 0

    lax.fori_loop(0, CV, h0, 0, unroll=2)

    for p in range(NPASS):
      src, dst = (bufa, bufb) if p % 2 == 0 else (bufb, bufa)
      shift = 8 * p
      scan_hist()
      final = p == NPASS - 1

      def perm(i, _, src=src, dst=dst, shift=shift, final=final):
        @plsc.parallel_loop(0, C, unroll=C)
        def _(c):
          k = src[pl.ds(c * CLEN + i * 16, 16)]
          d = _digit(k, shift) + c * RADIX
          cnt, last = plsc.scan_count(d)
          pos = plsc.load_gather(offs, [d]) + cnt
          plsc.store_scatter(offs, [d], pos, mask=last)
          pos = pos - 1
          if final:
            out = k ^ (~lax.shift_right_arithmetic(k, 31) | SIGN)
          else:
            out = k
          plsc.store_scatter(dst, [pos], out)

        return 0

      lax.fori_loop(0, CV, perm, 0, unroll=2)

      if not final:
        zero_hist()
        shift2 = 8 * (p + 1)

        def hist_next(i, _, dst=dst, shift2=shift2):
          @plsc.parallel_loop(0, C, unroll=C)
          def _(c):
            k = dst[pl.ds(c * CLEN + i * 16, 16)]
            d = _digit(k, shift2) + c * RADIX
            cnt, last = plsc.scan_count(d)
            plsc.addupdate_scatter(hist, [d], cnt, mask=last)

          return 0

        lax.fori_loop(0, CV, hist_next, 0, unroll=2)

    final_buf = bufb if NPASS % 2 == 1 else bufa
    pltpu.sync_copy(final_buf, out_hbm.at[row])
    return 0

  lax.fori_loop(wid * RPW, (wid + 1) * RPW, do_row, 0)


@jax.jit
def kernel(x):
  mesh = plsc.VectorSubcoreMesh(
      core_axis_name="c", subcore_axis_name="s", num_cores=NC, num_subcores=NS
  )
  run = pl.kernel(
      _sort_body,
      out_type=jax.ShapeDtypeStruct((ROWS, N), jnp.int32),
      mesh=mesh,
      scratch_types=[
          pltpu.VMEM((N,), jnp.int32),
          pltpu.VMEM((N,), jnp.int32),
          pltpu.VMEM((C * RADIX,), jnp.int32),
          pltpu.VMEM((C * RADIX,), jnp.int32),
      ],
      compiler_params=pltpu.CompilerParams(needs_layout_passes=False),
  )
  out_i32 = run(lax.bitcast_convert_type(x, jnp.int32))
  return lax.bitcast_convert_type(out_i32, jnp.float32)


# speedup vs baseline: 2.3933x; 1.2095x over previous
import functools

import jax
import jax.numpy as jnp
from jax import lax
from jax.experimental import pallas as pl
from jax.experimental.pallas import tpu as pltpu
from jax.experimental.pallas import tpu_sc as plsc

ROWS = 128
N = 32768
NC = 2
NS = 16
NW = NC * NS
RPW = ROWS // NW
NV = N // 16
RADIX = 256
NPASS = 4
C = 8
CLEN = N // C
CV = NV // C
SIGN = -2147483648


def _digit(k, shift):
  return lax.shift_right_logical(k, shift) & (RADIX - 1)


def _sort_body(x_hbm, out_hbm, bufa, bufb, hist, offs):
  wid = lax.axis_index("s") * NC + lax.axis_index("c")

  def zero_hist():
    z = jnp.zeros((16,), jnp.int32)

    def zh(i, _):
      hist[pl.ds(i * 16, 16)] = z
      return 0

    lax.fori_loop(0, C * RADIX // 16, zh, 0)

  def scan_hist():
    carry = jnp.int32(0)
    for v in range(RADIX // 16):
      hs = [hist[pl.ds(c * RADIX + v * 16, 16)] for c in range(C)]
      tot = hs[0]
      for c in range(1, C):
        tot = tot + hs[c]
      run = plsc.cumsum(tot) - tot + carry
      carry = carry + jnp.sum(tot)
      for c in range(C):
        offs[pl.ds(c * RADIX + v * 16, 16)] = run
        run = run + hs[c]

  def do_row(row, _):
    pltpu.sync_copy(x_hbm.at[row], bufa)

    zero_hist()

    def h0(i, _):
      @plsc.parallel_loop(0, C, unroll=C)
      def _(c):
        k = bufa[pl.ds(c * CLEN + i * 16, 16)]
        k = k ^ (lax.shift_right_arithmetic(k, 31) | SIGN)
        bufa[pl.ds(c * CLEN + i * 16, 16)] = k
        d = _digit(k, 0) + c * RADIX
        cnt, last = plsc.scan_count(d)
        plsc.addupdate_scatter(hist, [d], cnt, mask=last)

      return 0

    lax.fori_loop(0, CV, h0, 0)

    for p in range(NPASS):
      src, dst = (bufa, bufb) if p % 2 == 0 else (bufb, bufa)
      shift = 8 * p
      scan_hist()
      final = p == NPASS - 1

      def perm(i, _, src=src, dst=dst, shift=shift, final=final):
        @plsc.parallel_loop(0, C, unroll=C)
        def _(c):
          k = src[pl.ds(c * CLEN + i * 16, 16)]
          d = _digit(k, shift) + c * RADIX
          cnt, last = plsc.scan_count(d)
          pos = plsc.load_gather(offs, [d]) + cnt
          plsc.store_scatter(offs, [d], pos, mask=last)
          pos = pos - 1
          if final:
            out = k ^ (~lax.shift_right_arithmetic(k, 31) | SIGN)
          else:
            out = k
          plsc.store_scatter(dst, [pos], out)

        return 0

      lax.fori_loop(0, CV, perm, 0)

      if not final:
        zero_hist()
        shift2 = 8 * (p + 1)

        def hist_next(i, _, dst=dst, shift2=shift2):
          @plsc.parallel_loop(0, C, unroll=C)
          def _(c):
            k = dst[pl.ds(c * CLEN + i * 16, 16)]
            d = _digit(k, shift2) + c * RADIX
            cnt, last = plsc.scan_count(d)
            plsc.addupdate_scatter(hist, [d], cnt, mask=last)

          return 0

        lax.fori_loop(0, CV, hist_next, 0)

    final_buf = bufb if NPASS % 2 == 1 else bufa
    pltpu.sync_copy(final_buf, out_hbm.at[row])
    return 0

  lax.fori_loop(wid * RPW, (wid + 1) * RPW, do_row, 0)


@jax.jit
def kernel(x):
  mesh = plsc.VectorSubcoreMesh(
      core_axis_name="c", subcore_axis_name="s", num_cores=NC, num_subcores=NS
  )
  run = pl.kernel(
      _sort_body,
      out_type=jax.ShapeDtypeStruct((ROWS, N), jnp.int32),
      mesh=mesh,
      scratch_types=[
          pltpu.VMEM((N,), jnp.int32),
          pltpu.VMEM((N,), jnp.int32),
          pltpu.VMEM((C * RADIX,), jnp.int32),
          pltpu.VMEM((C * RADIX,), jnp.int32),
      ],
      compiler_params=pltpu.CompilerParams(needs_layout_passes=False),
  )
  out_i32 = run(lax.bitcast_convert_type(x, jnp.int32))
  return lax.bitcast_convert_type(out_i32, jnp.float32)


# speedup vs baseline: 2.5816x; 1.0787x over previous
import functools

import jax
import jax.numpy as jnp
from jax import lax
from jax.experimental import pallas as pl
from jax.experimental.pallas import tpu as pltpu
from jax.experimental.pallas import tpu_sc as plsc

ROWS = 128
N = 32768
NC = 2
NS = 16
NW = NC * NS
RPW = ROWS // NW
NV = N // 16
RADIX = 2048
SHIFTS = (0, 11, 22)
MASKS = (2047, 2047, 1023)
NPASS = 3
C = 8
CLEN = N // C
CV = NV // C
RV = RADIX // 16
SIGN = -2147483648


def _sort_body(x_hbm, out_hbm, bufa, bufb, hist, offs, tot):
  wid = lax.axis_index("s") * NC + lax.axis_index("c")

  def scan_hist():
    @plsc.parallel_loop(0, RV, unroll=8)
    def _(v):
      t = hist[pl.ds(v * 16, 16)]
      for c in range(1, C):
        t = t + hist[pl.ds(c * RADIX + v * 16, 16)]
      tot[pl.ds(v * 16, 16)] = t

    def excl(v, carry):
      t = tot[pl.ds(v * 16, 16)]
      tot[pl.ds(v * 16, 16)] = plsc.cumsum(t) - t + carry
      return carry + jnp.sum(t)

    lax.fori_loop(0, RV, excl, jnp.int32(0))

    z = jnp.zeros((16,), jnp.int32)

    @plsc.parallel_loop(0, RV, unroll=4)
    def _(v):
      run = tot[pl.ds(v * 16, 16)]
      for c in range(C):
        offs[pl.ds(c * RADIX + v * 16, 16)] = run
        run = run + hist[pl.ds(c * RADIX + v * 16, 16)]
        hist[pl.ds(c * RADIX + v * 16, 16)] = z

  def do_row(row, _):
    pltpu.sync_copy(x_hbm.at[row], bufa)

    def h0(i, _):
      @plsc.parallel_loop(0, C, unroll=C)
      def _(c):
        k = bufa[pl.ds(c * CLEN + i * 16, 16)]
        k = k ^ (lax.shift_right_arithmetic(k, 31) | SIGN)
        bufa[pl.ds(c * CLEN + i * 16, 16)] = k
        d = (k & MASKS[0]) + c * RADIX
        cnt, last = plsc.scan_count(d)
        plsc.addupdate_scatter(hist, [d], cnt, mask=last)

      return 0

    lax.fori_loop(0, CV, h0, 0)

    for p in range(NPASS):
      src, dst = (bufa, bufb) if p % 2 == 0 else (bufb, bufa)
      shift, mask = SHIFTS[p], MASKS[p]
      scan_hist()
      final = p == NPASS - 1

      def perm(i, _, src=src, dst=dst, shift=shift, mask=mask, final=final):
        @plsc.parallel_loop(0, C, unroll=C)
        def _(c):
          k = src[pl.ds(c * CLEN + i * 16, 16)]
          d = (lax.shift_right_logical(k, shift) & mask) + c * RADIX
          cnt, last = plsc.scan_count(d)
          pos = plsc.load_gather(offs, [d]) + cnt
          plsc.store_scatter(offs, [d], pos, mask=last)
          pos = pos - 1
          if final:
            out = k ^ (~lax.shift_right_arithmetic(k, 31) | SIGN)
          else:
            out = k
          plsc.store_scatter(dst, [pos], out)

        return 0

      lax.fori_loop(0, CV, perm, 0)

      if not final:
        shift2, mask2 = SHIFTS[p + 1], MASKS[p + 1]

        def hist_next(i, _, dst=dst, shift2=shift2, mask2=mask2):
          @plsc.parallel_loop(0, C, unroll=C)
          def _(c):
            k = dst[pl.ds(c * CLEN + i * 16, 16)]
            d = (lax.shift_right_logical(k, shift2) & mask2) + c * RADIX
            cnt, last = plsc.scan_count(d)
            plsc.addupdate_scatter(hist, [d], cnt, mask=last)

          return 0

        lax.fori_loop(0, CV, hist_next, 0)

    final_buf = bufb if NPASS % 2 == 1 else bufa
    pltpu.sync_copy(final_buf, out_hbm.at[row])
    return 0

  @plsc.parallel_loop(0, C * RV, unroll=8)
  def _(i):
    hist[pl.ds(i * 16, 16)] = jnp.zeros((16,), jnp.int32)

  lax.fori_loop(wid * RPW, (wid + 1) * RPW, do_row, 0)


@jax.jit
def kernel(x):
  mesh = plsc.VectorSubcoreMesh(
      core_axis_name="c", subcore_axis_name="s", num_cores=NC, num_subcores=NS
  )
  run = pl.kernel(
      _sort_body,
      out_type=jax.ShapeDtypeStruct((ROWS, N), jnp.int32),
      mesh=mesh,
      scratch_types=[
          pltpu.VMEM((N,), jnp.int32),
          pltpu.VMEM((N,), jnp.int32),
          pltpu.VMEM((C * RADIX,), jnp.int32),
          pltpu.VMEM((C * RADIX,), jnp.int32),
          pltpu.VMEM((RADIX,), jnp.int32),
      ],
      compiler_params=pltpu.CompilerParams(needs_layout_passes=False),
  )
  out_i32 = run(lax.bitcast_convert_type(x, jnp.int32))
  return lax.bitcast_convert_type(out_i32, jnp.float32)


# speedup vs baseline: 2.7127x; 1.0508x over previous
import functools

import jax
import jax.numpy as jnp
from jax import lax
from jax.experimental import pallas as pl
from jax.experimental.pallas import tpu as pltpu
from jax.experimental.pallas import tpu_sc as plsc

ROWS = 128
N = 32768
NC = 2
NS = 16
NW = NC * NS
RPW = ROWS // NW
NV = N // 16
RADIX = 2048
SHIFTS = (0, 11, 22)
MASKS = (2047, 2047, 1023)
NPASS = 3
C = 8
CLEN = N // C
CV = NV // C
RV = RADIX // 16
SIGN = -2147483648


def _sort_body(x_hbm, out_hbm, bufa, bufb, hist, offs, tot, gt):
  wid = lax.axis_index("s") * NC + lax.axis_index("c")

  def scan_hist():
    lane0 = lax.iota(jnp.int32, 16) == 0

    @plsc.parallel_loop(0, RV, unroll=8)
    def _(v):
      t = hist[pl.ds(v * 16, 16)]
      for c in range(1, C):
        t = t + hist[pl.ds(c * RADIX + v * 16, 16)]
      tot[pl.ds(v * 16, 16)] = t
      plsc.store_scatter(
          gt, [jnp.broadcast_to(v, (16,))],
          jnp.broadcast_to(jnp.sum(t), (16,)), mask=lane0)

    def excl(g, carry):
      t = gt[pl.ds(g * 16, 16)]
      gt[pl.ds(g * 16, 16)] = plsc.cumsum(t) - t + carry
      return carry + jnp.sum(t)

    lax.fori_loop(0, RV // 16, excl, jnp.int32(0))

    @plsc.parallel_loop(0, RV, unroll=8)
    def _(v):
      t = tot[pl.ds(v * 16, 16)]
      base = plsc.load_gather(gt, [jnp.broadcast_to(v, (16,))])
      tot[pl.ds(v * 16, 16)] = plsc.cumsum(t) - t + base

    z = jnp.zeros((16,), jnp.int32)

    @plsc.parallel_loop(0, RV, unroll=4)
    def _(v):
      run = tot[pl.ds(v * 16, 16)]
      for c in range(C):
        offs[pl.ds(c * RADIX + v * 16, 16)] = run
        run = run + hist[pl.ds(c * RADIX + v * 16, 16)]
        hist[pl.ds(c * RADIX + v * 16, 16)] = z

  def do_row(row, _):
    pltpu.sync_copy(x_hbm.at[row], bufa)

    def h0(i, _):
      @plsc.parallel_loop(0, C, unroll=C)
      def _(c):
        k = bufa[pl.ds(c * CLEN + i * 16, 16)]
        k = k ^ (lax.shift_right_arithmetic(k, 31) | SIGN)
        bufa[pl.ds(c * CLEN + i * 16, 16)] = k
        d = (k & MASKS[0]) + c * RADIX
        cnt, last = plsc.scan_count(d)
        plsc.addupdate_scatter(hist, [d], cnt, mask=last)

      return 0

    lax.fori_loop(0, CV, h0, 0)

    for p in range(NPASS):
      src, dst = (bufa, bufb) if p % 2 == 0 else (bufb, bufa)
      shift, mask = SHIFTS[p], MASKS[p]
      scan_hist()
      final = p == NPASS - 1

      def perm(i, _, src=src, dst=dst, shift=shift, mask=mask, final=final):
        @plsc.parallel_loop(0, C, unroll=C)
        def _(c):
          k = src[pl.ds(c * CLEN + i * 16, 16)]
          d = (lax.shift_right_logical(k, shift) & mask) + c * RADIX
          cnt, last = plsc.scan_count(d)
          pos = plsc.load_gather(offs, [d]) + cnt
          plsc.store_scatter(offs, [d], pos, mask=last)
          pos = pos - 1
          if final:
            out = k ^ (~lax.shift_right_arithmetic(k, 31) | SIGN)
          else:
            out = k
          plsc.store_scatter(dst, [pos], out)

        return 0

      lax.fori_loop(0, CV, perm, 0)

      if not final:
        shift2, mask2 = SHIFTS[p + 1], MASKS[p + 1]

        def hist_next(i, _, dst=dst, shift2=shift2, mask2=mask2):
          @plsc.parallel_loop(0, C, unroll=C)
          def _(c):
            k = dst[pl.ds(c * CLEN + i * 16, 16)]
            d = (lax.shift_right_logical(k, shift2) & mask2) + c * RADIX
            cnt, last = plsc.scan_count(d)
            plsc.addupdate_scatter(hist, [d], cnt, mask=last)

          return 0

        lax.fori_loop(0, CV, hist_next, 0)

    final_buf = bufb if NPASS % 2 == 1 else bufa
    pltpu.sync_copy(final_buf, out_hbm.at[row])
    return 0

  @plsc.parallel_loop(0, C * RV, unroll=8)
  def _(i):
    hist[pl.ds(i * 16, 16)] = jnp.zeros((16,), jnp.int32)

  lax.fori_loop(wid * RPW, (wid + 1) * RPW, do_row, 0)


@jax.jit
def kernel(x):
  mesh = plsc.VectorSubcoreMesh(
      core_axis_name="c", subcore_axis_name="s", num_cores=NC, num_subcores=NS
  )
  run = pl.kernel(
      _sort_body,
      out_type=jax.ShapeDtypeStruct((ROWS, N), jnp.int32),
      mesh=mesh,
      scratch_types=[
          pltpu.VMEM((N,), jnp.int32),
          pltpu.VMEM((N,), jnp.int32),
          pltpu.VMEM((C * RADIX,), jnp.int32),
          pltpu.VMEM((C * RADIX,), jnp.int32),
          pltpu.VMEM((RADIX,), jnp.int32),
          pltpu.VMEM((RV,), jnp.int32),
      ],
      compiler_params=pltpu.CompilerParams(needs_layout_passes=False),
  )
  out_i32 = run(lax.bitcast_convert_type(x, jnp.int32))
  return lax.bitcast_convert_type(out_i32, jnp.float32)
